# baseline (speedup 1.0000x reference)
import jax
import jax.numpy as jnp
from jax.experimental import pallas as pl

_FIXED = jnp.array([
    [0.0, 0.0, 0.0, 0.0],
    [1.0, 0.0, 0.0, 0.0],
    [0.0, 1.0, 0.0, 0.0],
    [0.0, 0.0, 1.0, 0.0],
    [0.0, 0.0, 0.0, 1.0],
    [0.25, 0.25, 0.25, 0.25],
], dtype=jnp.float32)


def _concat_kernel(fixed_ref, w_ref, out_ref):
    out_ref[...] = jnp.concatenate([fixed_ref[...], w_ref[...]], axis=0)


def kernel(RNA_embedding_weight):
    return pl.pallas_call(
        _concat_kernel,
        out_shape=jax.ShapeDtypeStruct((19, 4), jnp.float32),
    )(_FIXED, RNA_embedding_weight)
